# baseline (speedup 1.0000x reference)
import functools

import jax
import jax.numpy as jnp
from jax import lax
from jax.experimental import pallas as pl
from jax.experimental.pallas import tpu as pltpu
from jax.experimental.pallas import tpu_sc as plsc

N = 10000
E = 320000
H = 128
C = 16
G = 64
L = 3

NC = 2
NS = 16
NW = NC * NS
K = 128
EPW = 10240
E2 = NW * EPW
EPAD = E2 - E
NCH = EPW // K
NPAIR = NCH // 2
RPT = 632
N2 = NS * RPT
ZR = 32

BN_SCALE = 1.0 / (1.0 + 1e-5) ** 0.5


def _seg_sum_body(h_hbm, src_hbm, dst_hbm, out0_hbm, out1_hbm,
                  sidx0, didx0, sidx1, didx1, rows0, rows1, zbuf, acc,
                  semi0, semi1, semg0, semg1):
    c = lax.axis_index("c")
    s = lax.axis_index("s")
    wid = c * NS + s

    zeros16 = jnp.zeros((16,), jnp.float32)

    def zrow(i, _):
        def zcol(j, _):
            zbuf[i, pl.ds(j * 16, 16)] = zeros16
            return 0
        return lax.fori_loop(0, H // 16, zcol, 0)

    base = wid * EPW

    def start_idx(off, sbuf, dbuf, sem):
        pltpu.async_copy(src_hbm.at[pl.ds(off, K)], sbuf, sem)
        pltpu.async_copy(dst_hbm.at[pl.ds(off, K)], dbuf, sem)

    def wait_idx(off, sbuf, dbuf, sem):
        pltpu.make_async_copy(src_hbm.at[pl.ds(off, K)], sbuf, sem).wait()
        pltpu.make_async_copy(dst_hbm.at[pl.ds(off, K)], dbuf, sem).wait()

    lax.fori_loop(0, ZR, zrow, 0)
    for r in range(RPT // ZR):
        pltpu.sync_copy(zbuf, acc.at[pl.ds(s * RPT + r * ZR, ZR)])
    rem = RPT % ZR
    if rem:
        pltpu.sync_copy(zbuf.at[pl.ds(0, rem)],
                        acc.at[pl.ds(s * RPT + (RPT // ZR) * ZR, rem)])
    plsc.subcore_barrier()

    start_idx(base, sidx0, didx0, semi0)
    wait_idx(base, sidx0, didx0, semi0)
    pltpu.async_copy(h_hbm.at[sidx0], rows0, semg0)
    start_idx(base + K, sidx1, didx1, semi1)

    def pair(g, _):
        a = base + 2 * g * K
        b = a + K
        nxt = g < NPAIR - 1
        wait_idx(b, sidx1, didx1, semi1)
        pltpu.async_copy(h_hbm.at[sidx1], rows1, semg1)
        pltpu.make_async_copy(h_hbm.at[sidx0], rows0, semg0).wait()
        pltpu.sync_copy(rows0, acc.at[didx0], add=True)

        @pl.when(nxt)
        def _pref_a():
            start_idx(b + K, sidx0, didx0, semi0)

        pltpu.make_async_copy(h_hbm.at[sidx1], rows1, semg1).wait()

        @pl.when(nxt)
        def _gather_a():
            wait_idx(b + K, sidx0, didx0, semi0)
            pltpu.async_copy(h_hbm.at[sidx0], rows0, semg0)

        pltpu.sync_copy(rows1, acc.at[didx1], add=True)

        @pl.when(nxt)
        def _pref_b():
            start_idx(b + 2 * K, sidx1, didx1, semi1)

        return 0

    lax.fori_loop(0, NPAIR, pair, 0)
    plsc.subcore_barrier()

    @pl.when(c == 0)
    def _w0():
        pltpu.sync_copy(acc.at[pl.ds(s * RPT, RPT)],
                        out0_hbm.at[pl.ds(s * RPT, RPT)])

    @pl.when(c == 1)
    def _w1():
        pltpu.sync_copy(acc.at[pl.ds(s * RPT, RPT)],
                        out1_hbm.at[pl.ds(s * RPT, RPT)])


@functools.cache
def _make_seg_sum():
    return pl.kernel(
        _seg_sum_body,
        out_type=[jax.ShapeDtypeStruct((N2, H), jnp.float32),
                  jax.ShapeDtypeStruct((N2, H), jnp.float32)],
        mesh=plsc.VectorSubcoreMesh(core_axis_name="c", subcore_axis_name="s",
                                    num_cores=NC, num_subcores=NS),
        scratch_types=[
            pltpu.VMEM((K,), jnp.int32),
            pltpu.VMEM((K,), jnp.int32),
            pltpu.VMEM((K,), jnp.int32),
            pltpu.VMEM((K,), jnp.int32),
            pltpu.VMEM((K, H), jnp.float32),
            pltpu.VMEM((K, H), jnp.float32),
            pltpu.VMEM((ZR, H), jnp.float32),
            pltpu.VMEM_SHARED((N2, H), jnp.float32),
            pltpu.SemaphoreType.DMA,
            pltpu.SemaphoreType.DMA,
            pltpu.SemaphoreType.DMA,
            pltpu.SemaphoreType.DMA,
        ],
    )


def _seg_sum(h, src, dst):
    return _make_seg_sum()(h, src, dst)


def _pad_edges(src, dst):
    ar = jnp.arange(EPAD, dtype=jnp.int32)
    pad_src = ar % N
    pad_dst = N + ar % (N2 - N)
    return (jnp.concatenate([src, pad_src]),
            jnp.concatenate([dst, pad_dst]))


BT = 1000
NB = N // BT


def _dense_body(eps_ref, h_ref, p0_ref, p1_ref, wa_ref, ba_ref, wb_ref,
                bb_ref, g_ref, be_ref, o_ref):
    z = h_ref[...] * eps_ref[0] + p0_ref[...] + p1_ref[...]
    z = jnp.maximum(
        jnp.dot(z, wa_ref[...], preferred_element_type=jnp.float32)
        + ba_ref[...], 0.0)
    z = jnp.maximum(
        jnp.dot(z, wb_ref[...], preferred_element_type=jnp.float32)
        + bb_ref[...], 0.0)
    o_ref[...] = z * (g_ref[...] * BN_SCALE) + be_ref[...]


def _dense_layer(epsv, h, p0, p1, Wa, ba, Wb, bb, gam, bet):
    return pl.pallas_call(
        _dense_body,
        grid=(NB,),
        in_specs=[
            pl.BlockSpec(memory_space=pltpu.SMEM),
            pl.BlockSpec((BT, H), lambda i: (i, 0)),
            pl.BlockSpec((BT, H), lambda i: (i, 0)),
            pl.BlockSpec((BT, H), lambda i: (i, 0)),
            pl.BlockSpec((H, H), lambda i: (0, 0)),
            pl.BlockSpec((1, H), lambda i: (0, 0)),
            pl.BlockSpec((H, H), lambda i: (0, 0)),
            pl.BlockSpec((1, H), lambda i: (0, 0)),
            pl.BlockSpec((1, H), lambda i: (0, 0)),
            pl.BlockSpec((1, H), lambda i: (0, 0)),
        ],
        out_specs=pl.BlockSpec((BT, H), lambda i: (i, 0)),
        out_shape=jax.ShapeDtypeStruct((N, H), jnp.float32),
    )(epsv, h, p0, p1, Wa, ba.reshape(1, H), Wb, bb.reshape(1, H),
      gam.reshape(1, H), bet.reshape(1, H))


def _pool_body(eps_ref, h_ref, p0_ref, p1_ref, wa_ref, ba_ref, wb_ref,
               bb_ref, g_ref, be_ref, batch_ref, wl1_ref, bl1_ref, wl2_ref,
               bl2_ref, o_ref, sums_ref, cnts_ref):
    i = pl.program_id(0)

    @pl.when(i == 0)
    def _init():
        sums_ref[...] = jnp.zeros((G, H), jnp.float32)
        cnts_ref[...] = jnp.zeros((G, H), jnp.float32)

    z = h_ref[...] * eps_ref[0] + p0_ref[...] + p1_ref[...]
    z = jnp.maximum(
        jnp.dot(z, wa_ref[...], preferred_element_type=jnp.float32)
        + ba_ref[...], 0.0)
    z = jnp.maximum(
        jnp.dot(z, wb_ref[...], preferred_element_type=jnp.float32)
        + bb_ref[...], 0.0)
    h3 = z * (g_ref[...] * BN_SCALE) + be_ref[...]

    b = batch_ref[0, 0, :]
    oh = (b[:, None] == lax.broadcasted_iota(jnp.int32, (BT, G), 1))
    oh = oh.astype(jnp.float32)
    sums_ref[...] += lax.dot_general(
        oh, h3, (((0,), (0,)), ((), ())),
        preferred_element_type=jnp.float32)
    cnts_ref[...] += lax.dot_general(
        oh, jnp.ones((BT, H), jnp.float32), (((0,), (0,)), ((), ())),
        preferred_element_type=jnp.float32)

    @pl.when(i == NB - 1)
    def _final():
        pooled = sums_ref[...] / jnp.maximum(cnts_ref[...], 1.0)
        o1 = jnp.maximum(
            jnp.dot(pooled, wl1_ref[...], preferred_element_type=jnp.float32)
            + bl1_ref[...], 0.0)
        o2 = (jnp.dot(o1, wl2_ref[...], preferred_element_type=jnp.float32)
              + bl2_ref[...])
        m = jnp.max(o2, axis=1, keepdims=True)
        ex = jnp.exp(o2 - m)
        lse = jnp.log(jnp.sum(ex, axis=1, keepdims=True))
        o_ref[...] = o2 - m - lse


def _pool_head(epsv, h, p0, p1, Wa, ba, Wb, bb, gam, bet,
               batch3d, Wl1, bl1, Wl2, bl2):
    return pl.pallas_call(
        _pool_body,
        grid=(NB,),
        in_specs=[
            pl.BlockSpec(memory_space=pltpu.SMEM),
            pl.BlockSpec((BT, H), lambda i: (i, 0)),
            pl.BlockSpec((BT, H), lambda i: (i, 0)),
            pl.BlockSpec((BT, H), lambda i: (i, 0)),
            pl.BlockSpec((H, H), lambda i: (0, 0)),
            pl.BlockSpec((1, H), lambda i: (0, 0)),
            pl.BlockSpec((H, H), lambda i: (0, 0)),
            pl.BlockSpec((1, H), lambda i: (0, 0)),
            pl.BlockSpec((1, H), lambda i: (0, 0)),
            pl.BlockSpec((1, H), lambda i: (0, 0)),
            pl.BlockSpec((1, 1, BT), lambda i: (i, 0, 0)),
            pl.BlockSpec((H, H), lambda i: (0, 0)),
            pl.BlockSpec((1, H), lambda i: (0, 0)),
            pl.BlockSpec((H, C), lambda i: (0, 0)),
            pl.BlockSpec((1, C), lambda i: (0, 0)),
        ],
        out_specs=pl.BlockSpec((G, C), lambda i: (0, 0)),
        out_shape=jax.ShapeDtypeStruct((G, C), jnp.float32),
        scratch_shapes=[
            pltpu.VMEM((G, H), jnp.float32),
            pltpu.VMEM((G, H), jnp.float32),
        ],
    )(epsv, h, p0, p1, Wa, ba.reshape(1, H), Wb, bb.reshape(1, H),
      gam.reshape(1, H), bet.reshape(1, H), batch3d, Wl1, bl1, Wl2, bl2)


def kernel(x, edge_index, batch, eps, Wa, ba, Wb, bb, gam, bet,
           Wl1, bl1, Wl2, bl2):
    src, dst = _pad_edges(edge_index[0], edge_index[1])
    batch3d = batch.reshape(NB, 1, BT)
    h = x
    for i in range(L - 1):
        p0, p1 = _seg_sum(h, src, dst)
        h = _dense_layer((1.0 + eps[i]).reshape(1), h, p0, p1,
                         Wa[i], ba[i], Wb[i], bb[i], gam[i], bet[i])
    p0, p1 = _seg_sum(h, src, dst)
    return _pool_head((1.0 + eps[L - 1]).reshape(1), h, p0, p1,
                      Wa[L - 1], ba[L - 1], Wb[L - 1], bb[L - 1],
                      gam[L - 1], bet[L - 1], batch3d,
                      Wl1, bl1.reshape(1, H), Wl2, bl2.reshape(1, C))


# speedup vs baseline: 1.0237x; 1.0237x over previous
import functools

import jax
import jax.numpy as jnp
from jax import lax
from jax.experimental import pallas as pl
from jax.experimental.pallas import tpu as pltpu
from jax.experimental.pallas import tpu_sc as plsc

N = 10000
E = 320000
H = 128
C = 16
G = 64
L = 3

NC = 2
NS = 16
NW = NC * NS
K = 128
EPW = 10240
E2 = NW * EPW
EPAD = E2 - E
NCH = EPW // K
NPAIR = NCH // 2
RPT = 632
N2 = NS * RPT
ZR = 32

BN_SCALE = 1.0 / (1.0 + 1e-5) ** 0.5


def _seg_sum_body(h_hbm, src_hbm, dst_hbm, out0_hbm, out1_hbm,
                  sidx0, didx0, sidx1, didx1, rows0, rows1, zbuf, acc,
                  semi0, semi1, semg0, semg1):
    c = lax.axis_index("c")
    s = lax.axis_index("s")
    wid = c * NS + s

    zeros16 = jnp.zeros((16,), jnp.float32)

    def zrow(i, _):
        def zcol(j, _):
            zbuf[i, pl.ds(j * 16, 16)] = zeros16
            return 0
        return lax.fori_loop(0, H // 16, zcol, 0)

    base = wid * EPW

    def start_idx(off, sbuf, dbuf, sem):
        pltpu.async_copy(src_hbm.at[pl.ds(off, K)], sbuf, sem)
        pltpu.async_copy(dst_hbm.at[pl.ds(off, K)], dbuf, sem)

    def wait_idx(off, sbuf, dbuf, sem):
        pltpu.make_async_copy(src_hbm.at[pl.ds(off, K)], sbuf, sem).wait()
        pltpu.make_async_copy(dst_hbm.at[pl.ds(off, K)], dbuf, sem).wait()

    lax.fori_loop(0, ZR, zrow, 0)
    for r in range(RPT // ZR):
        pltpu.sync_copy(zbuf, acc.at[pl.ds(s * RPT + r * ZR, ZR)])
    rem = RPT % ZR
    if rem:
        pltpu.sync_copy(zbuf.at[pl.ds(0, rem)],
                        acc.at[pl.ds(s * RPT + (RPT // ZR) * ZR, rem)])
    plsc.subcore_barrier()

    start_idx(base, sidx0, didx0, semi0)
    wait_idx(base, sidx0, didx0, semi0)
    pltpu.async_copy(h_hbm.at[sidx0], rows0, semg0)
    start_idx(base + K, sidx1, didx1, semi1)

    def pair(g, _):
        a = base + 2 * g * K
        b = a + K
        nxt = g < NPAIR - 1
        wait_idx(b, sidx1, didx1, semi1)
        pltpu.async_copy(h_hbm.at[sidx1], rows1, semg1)
        pltpu.make_async_copy(h_hbm.at[sidx0], rows0, semg0).wait()
        pltpu.sync_copy(rows0, acc.at[didx0], add=True)

        @pl.when(nxt)
        def _pref_a():
            start_idx(b + K, sidx0, didx0, semi0)

        pltpu.make_async_copy(h_hbm.at[sidx1], rows1, semg1).wait()

        @pl.when(nxt)
        def _gather_a():
            wait_idx(b + K, sidx0, didx0, semi0)
            pltpu.async_copy(h_hbm.at[sidx0], rows0, semg0)

        pltpu.sync_copy(rows1, acc.at[didx1], add=True)

        @pl.when(nxt)
        def _pref_b():
            start_idx(b + 2 * K, sidx1, didx1, semi1)

        return 0

    lax.fori_loop(0, NPAIR, pair, 0)
    plsc.subcore_barrier()

    @pl.when(c == 0)
    def _w0():
        pltpu.sync_copy(acc.at[pl.ds(s * RPT, RPT)],
                        out0_hbm.at[pl.ds(s * RPT, RPT)])

    @pl.when(c == 1)
    def _w1():
        pltpu.sync_copy(acc.at[pl.ds(s * RPT, RPT)],
                        out1_hbm.at[pl.ds(s * RPT, RPT)])


@functools.cache
def _make_seg_sum():
    return pl.kernel(
        _seg_sum_body,
        out_type=[jax.ShapeDtypeStruct((N2, H), jnp.float32),
                  jax.ShapeDtypeStruct((N2, H), jnp.float32)],
        mesh=plsc.VectorSubcoreMesh(core_axis_name="c", subcore_axis_name="s",
                                    num_cores=NC, num_subcores=NS),
        scratch_types=[
            pltpu.VMEM((K,), jnp.int32),
            pltpu.VMEM((K,), jnp.int32),
            pltpu.VMEM((K,), jnp.int32),
            pltpu.VMEM((K,), jnp.int32),
            pltpu.VMEM((K, H), jnp.float32),
            pltpu.VMEM((K, H), jnp.float32),
            pltpu.VMEM((ZR, H), jnp.float32),
            pltpu.VMEM_SHARED((N2, H), jnp.float32),
            pltpu.SemaphoreType.DMA,
            pltpu.SemaphoreType.DMA,
            pltpu.SemaphoreType.DMA,
            pltpu.SemaphoreType.DMA,
        ],
    )


def _seg_sum(h, src, dst):
    return _make_seg_sum()(h, src, dst)


def _pad_edges(src, dst):
    ar = jnp.arange(EPAD, dtype=jnp.int32)
    pad_src = ar % N
    pad_dst = N + ar % (N2 - N)
    return (jnp.concatenate([src, pad_src]),
            jnp.concatenate([dst, pad_dst]))


BT = 2000
NB = N // BT


def _dense_body(eps_ref, h_ref, p0_ref, p1_ref, wa_ref, ba_ref, wb_ref,
                bb_ref, g_ref, be_ref, o_ref):
    z = h_ref[...] * eps_ref[0] + p0_ref[...] + p1_ref[...]
    z = jnp.maximum(
        jnp.dot(z, wa_ref[...], preferred_element_type=jnp.float32)
        + ba_ref[...], 0.0)
    z = jnp.maximum(
        jnp.dot(z, wb_ref[...], preferred_element_type=jnp.float32)
        + bb_ref[...], 0.0)
    o_ref[...] = z * (g_ref[...] * BN_SCALE) + be_ref[...]


def _dense_layer(epsv, h, p0, p1, Wa, ba, Wb, bb, gam, bet):
    return pl.pallas_call(
        _dense_body,
        grid=(NB,),
        in_specs=[
            pl.BlockSpec(memory_space=pltpu.SMEM),
            pl.BlockSpec((BT, H), lambda i: (i, 0)),
            pl.BlockSpec((BT, H), lambda i: (i, 0)),
            pl.BlockSpec((BT, H), lambda i: (i, 0)),
            pl.BlockSpec((H, H), lambda i: (0, 0)),
            pl.BlockSpec((1, H), lambda i: (0, 0)),
            pl.BlockSpec((H, H), lambda i: (0, 0)),
            pl.BlockSpec((1, H), lambda i: (0, 0)),
            pl.BlockSpec((1, H), lambda i: (0, 0)),
            pl.BlockSpec((1, H), lambda i: (0, 0)),
        ],
        out_specs=pl.BlockSpec((BT, H), lambda i: (i, 0)),
        out_shape=jax.ShapeDtypeStruct((N, H), jnp.float32),
    )(epsv, h, p0, p1, Wa, ba.reshape(1, H), Wb, bb.reshape(1, H),
      gam.reshape(1, H), bet.reshape(1, H))


def _pool_body(eps_ref, h_ref, p0_ref, p1_ref, wa_ref, ba_ref, wb_ref,
               bb_ref, g_ref, be_ref, batch_ref, wl1_ref, bl1_ref, wl2_ref,
               bl2_ref, o_ref, sums_ref, cnts_ref):
    i = pl.program_id(0)

    @pl.when(i == 0)
    def _init():
        sums_ref[...] = jnp.zeros((G, H), jnp.float32)
        cnts_ref[...] = jnp.zeros((G, H), jnp.float32)

    z = h_ref[...] * eps_ref[0] + p0_ref[...] + p1_ref[...]
    z = jnp.maximum(
        jnp.dot(z, wa_ref[...], preferred_element_type=jnp.float32)
        + ba_ref[...], 0.0)
    z = jnp.maximum(
        jnp.dot(z, wb_ref[...], preferred_element_type=jnp.float32)
        + bb_ref[...], 0.0)
    h3 = z * (g_ref[...] * BN_SCALE) + be_ref[...]

    b = batch_ref[0, 0, :]
    oh = (b[:, None] == lax.broadcasted_iota(jnp.int32, (BT, G), 1))
    oh = oh.astype(jnp.float32)
    sums_ref[...] += lax.dot_general(
        oh, h3, (((0,), (0,)), ((), ())),
        preferred_element_type=jnp.float32)
    cnts_ref[...] += lax.dot_general(
        oh, jnp.ones((BT, H), jnp.float32), (((0,), (0,)), ((), ())),
        preferred_element_type=jnp.float32)

    @pl.when(i == NB - 1)
    def _final():
        pooled = sums_ref[...] / jnp.maximum(cnts_ref[...], 1.0)
        o1 = jnp.maximum(
            jnp.dot(pooled, wl1_ref[...], preferred_element_type=jnp.float32)
            + bl1_ref[...], 0.0)
        o2 = (jnp.dot(o1, wl2_ref[...], preferred_element_type=jnp.float32)
              + bl2_ref[...])
        m = jnp.max(o2, axis=1, keepdims=True)
        ex = jnp.exp(o2 - m)
        lse = jnp.log(jnp.sum(ex, axis=1, keepdims=True))
        o_ref[...] = o2 - m - lse


def _pool_head(epsv, h, p0, p1, Wa, ba, Wb, bb, gam, bet,
               batch3d, Wl1, bl1, Wl2, bl2):
    return pl.pallas_call(
        _pool_body,
        grid=(NB,),
        in_specs=[
            pl.BlockSpec(memory_space=pltpu.SMEM),
            pl.BlockSpec((BT, H), lambda i: (i, 0)),
            pl.BlockSpec((BT, H), lambda i: (i, 0)),
            pl.BlockSpec((BT, H), lambda i: (i, 0)),
            pl.BlockSpec((H, H), lambda i: (0, 0)),
            pl.BlockSpec((1, H), lambda i: (0, 0)),
            pl.BlockSpec((H, H), lambda i: (0, 0)),
            pl.BlockSpec((1, H), lambda i: (0, 0)),
            pl.BlockSpec((1, H), lambda i: (0, 0)),
            pl.BlockSpec((1, H), lambda i: (0, 0)),
            pl.BlockSpec((1, 1, BT), lambda i: (i, 0, 0)),
            pl.BlockSpec((H, H), lambda i: (0, 0)),
            pl.BlockSpec((1, H), lambda i: (0, 0)),
            pl.BlockSpec((H, C), lambda i: (0, 0)),
            pl.BlockSpec((1, C), lambda i: (0, 0)),
        ],
        out_specs=pl.BlockSpec((G, C), lambda i: (0, 0)),
        out_shape=jax.ShapeDtypeStruct((G, C), jnp.float32),
        scratch_shapes=[
            pltpu.VMEM((G, H), jnp.float32),
            pltpu.VMEM((G, H), jnp.float32),
        ],
    )(epsv, h, p0, p1, Wa, ba.reshape(1, H), Wb, bb.reshape(1, H),
      gam.reshape(1, H), bet.reshape(1, H), batch3d, Wl1, bl1, Wl2, bl2)


def kernel(x, edge_index, batch, eps, Wa, ba, Wb, bb, gam, bet,
           Wl1, bl1, Wl2, bl2):
    src, dst = _pad_edges(edge_index[0], edge_index[1])
    batch3d = batch.reshape(NB, 1, BT)
    h = x
    for i in range(L - 1):
        p0, p1 = _seg_sum(h, src, dst)
        h = _dense_layer((1.0 + eps[i]).reshape(1), h, p0, p1,
                         Wa[i], ba[i], Wb[i], bb[i], gam[i], bet[i])
    p0, p1 = _seg_sum(h, src, dst)
    return _pool_head((1.0 + eps[L - 1]).reshape(1), h, p0, p1,
                      Wa[L - 1], ba[L - 1], Wb[L - 1], bb[L - 1],
                      gam[L - 1], bet[L - 1], batch3d,
                      Wl1, bl1.reshape(1, H), Wl2, bl2.reshape(1, C))


# speedup vs baseline: 1.0330x; 1.0091x over previous
import functools

import jax
import jax.numpy as jnp
from jax import lax
from jax.experimental import pallas as pl
from jax.experimental.pallas import tpu as pltpu
from jax.experimental.pallas import tpu_sc as plsc

N = 10000
E = 320000
H = 128
C = 16
G = 64
L = 3

NC = 2
NS = 16
NW = NC * NS
K = 128
EPW = 10240
E2 = NW * EPW
EPAD = E2 - E
NCH = EPW // K
NPAIR = NCH // 2
RPT = 632
N2 = NS * RPT
ZR = 32

BN_SCALE = 1.0 / (1.0 + 1e-5) ** 0.5


def _seg_sum_body(h_hbm, src_hbm, dst_hbm, out0_hbm, out1_hbm,
                  sidx0, didx0, sidx1, didx1, rows0, rows1, zbuf, acc,
                  semi0, semi1, semg0, semg1):
    c = lax.axis_index("c")
    s = lax.axis_index("s")
    wid = c * NS + s

    zeros16 = jnp.zeros((16,), jnp.float32)

    def zrow(i, _):
        def zcol(j, _):
            zbuf[i, pl.ds(j * 16, 16)] = zeros16
            return 0
        return lax.fori_loop(0, H // 16, zcol, 0)

    base = wid * EPW

    def start_idx(off, sbuf, dbuf, sem):
        pltpu.async_copy(src_hbm.at[pl.ds(off, K)], sbuf, sem)
        pltpu.async_copy(dst_hbm.at[pl.ds(off, K)], dbuf, sem)

    def wait_idx(off, sbuf, dbuf, sem):
        pltpu.make_async_copy(src_hbm.at[pl.ds(off, K)], sbuf, sem).wait()
        pltpu.make_async_copy(dst_hbm.at[pl.ds(off, K)], dbuf, sem).wait()

    lax.fori_loop(0, ZR, zrow, 0)
    for r in range(RPT // ZR):
        pltpu.sync_copy(zbuf, acc.at[pl.ds(s * RPT + r * ZR, ZR)])
    rem = RPT % ZR
    if rem:
        pltpu.sync_copy(zbuf.at[pl.ds(0, rem)],
                        acc.at[pl.ds(s * RPT + (RPT // ZR) * ZR, rem)])
    plsc.subcore_barrier()

    start_idx(base, sidx0, didx0, semi0)
    wait_idx(base, sidx0, didx0, semi0)
    pltpu.async_copy(h_hbm.at[sidx0], rows0, semg0)
    start_idx(base + K, sidx1, didx1, semi1)

    def pair(g, _):
        a = base + 2 * g * K
        b = a + K
        nxt = g < NPAIR - 1
        wait_idx(b, sidx1, didx1, semi1)
        pltpu.async_copy(h_hbm.at[sidx1], rows1, semg1)
        pltpu.make_async_copy(h_hbm.at[sidx0], rows0, semg0).wait()
        pltpu.sync_copy(rows0, acc.at[didx0], add=True)

        @pl.when(nxt)
        def _pref_a():
            start_idx(b + K, sidx0, didx0, semi0)

        pltpu.make_async_copy(h_hbm.at[sidx1], rows1, semg1).wait()

        @pl.when(nxt)
        def _gather_a():
            wait_idx(b + K, sidx0, didx0, semi0)
            pltpu.async_copy(h_hbm.at[sidx0], rows0, semg0)

        pltpu.sync_copy(rows1, acc.at[didx1], add=True)

        @pl.when(nxt)
        def _pref_b():
            start_idx(b + 2 * K, sidx1, didx1, semi1)

        return 0

    lax.fori_loop(0, NPAIR, pair, 0)
    plsc.subcore_barrier()

    @pl.when(c == 0)
    def _w0():
        pltpu.sync_copy(acc.at[pl.ds(s * RPT, RPT)],
                        out0_hbm.at[pl.ds(s * RPT, RPT)])

    @pl.when(c == 1)
    def _w1():
        pltpu.sync_copy(acc.at[pl.ds(s * RPT, RPT)],
                        out1_hbm.at[pl.ds(s * RPT, RPT)])


@functools.cache
def _make_seg_sum():
    return pl.kernel(
        _seg_sum_body,
        out_type=[jax.ShapeDtypeStruct((N2, H), jnp.float32),
                  jax.ShapeDtypeStruct((N2, H), jnp.float32)],
        mesh=plsc.VectorSubcoreMesh(core_axis_name="c", subcore_axis_name="s",
                                    num_cores=NC, num_subcores=NS),
        scratch_types=[
            pltpu.VMEM((K,), jnp.int32),
            pltpu.VMEM((K,), jnp.int32),
            pltpu.VMEM((K,), jnp.int32),
            pltpu.VMEM((K,), jnp.int32),
            pltpu.VMEM((K, H), jnp.float32),
            pltpu.VMEM((K, H), jnp.float32),
            pltpu.VMEM((ZR, H), jnp.float32),
            pltpu.VMEM_SHARED((N2, H), jnp.float32),
            pltpu.SemaphoreType.DMA,
            pltpu.SemaphoreType.DMA,
            pltpu.SemaphoreType.DMA,
            pltpu.SemaphoreType.DMA,
        ],
    )


def _seg_sum(h, src, dst):
    return _make_seg_sum()(h, src, dst)


def _pad_edges(src, dst):
    ar = jnp.arange(EPAD, dtype=jnp.int32)
    pad_src = ar % N
    pad_dst = N + ar % (N2 - N)
    return (jnp.concatenate([src, pad_src]),
            jnp.concatenate([dst, pad_dst]))


BT = 5000
NB = N // BT


def _dense_body(eps_ref, h_ref, p0_ref, p1_ref, wa_ref, ba_ref, wb_ref,
                bb_ref, g_ref, be_ref, o_ref):
    z = h_ref[...] * eps_ref[0] + p0_ref[...] + p1_ref[...]
    z = jnp.maximum(
        jnp.dot(z, wa_ref[...], preferred_element_type=jnp.float32)
        + ba_ref[...], 0.0)
    z = jnp.maximum(
        jnp.dot(z, wb_ref[...], preferred_element_type=jnp.float32)
        + bb_ref[...], 0.0)
    o_ref[...] = z * (g_ref[...] * BN_SCALE) + be_ref[...]


def _dense_layer(epsv, h, p0, p1, Wa, ba, Wb, bb, gam, bet):
    return pl.pallas_call(
        _dense_body,
        grid=(NB,),
        in_specs=[
            pl.BlockSpec(memory_space=pltpu.SMEM),
            pl.BlockSpec((BT, H), lambda i: (i, 0)),
            pl.BlockSpec((BT, H), lambda i: (i, 0)),
            pl.BlockSpec((BT, H), lambda i: (i, 0)),
            pl.BlockSpec((H, H), lambda i: (0, 0)),
            pl.BlockSpec((1, H), lambda i: (0, 0)),
            pl.BlockSpec((H, H), lambda i: (0, 0)),
            pl.BlockSpec((1, H), lambda i: (0, 0)),
            pl.BlockSpec((1, H), lambda i: (0, 0)),
            pl.BlockSpec((1, H), lambda i: (0, 0)),
        ],
        out_specs=pl.BlockSpec((BT, H), lambda i: (i, 0)),
        out_shape=jax.ShapeDtypeStruct((N, H), jnp.float32),
    )(epsv, h, p0, p1, Wa, ba.reshape(1, H), Wb, bb.reshape(1, H),
      gam.reshape(1, H), bet.reshape(1, H))


def _pool_body(eps_ref, h_ref, p0_ref, p1_ref, wa_ref, ba_ref, wb_ref,
               bb_ref, g_ref, be_ref, batch_ref, wl1_ref, bl1_ref, wl2_ref,
               bl2_ref, o_ref, sums_ref, cnts_ref):
    i = pl.program_id(0)

    @pl.when(i == 0)
    def _init():
        sums_ref[...] = jnp.zeros((G, H), jnp.float32)
        cnts_ref[...] = jnp.zeros((G, H), jnp.float32)

    z = h_ref[...] * eps_ref[0] + p0_ref[...] + p1_ref[...]
    z = jnp.maximum(
        jnp.dot(z, wa_ref[...], preferred_element_type=jnp.float32)
        + ba_ref[...], 0.0)
    z = jnp.maximum(
        jnp.dot(z, wb_ref[...], preferred_element_type=jnp.float32)
        + bb_ref[...], 0.0)
    h3 = z * (g_ref[...] * BN_SCALE) + be_ref[...]

    b = batch_ref[0, 0, :]
    oh = (b[:, None] == lax.broadcasted_iota(jnp.int32, (BT, G), 1))
    oh = oh.astype(jnp.float32)
    sums_ref[...] += lax.dot_general(
        oh, h3, (((0,), (0,)), ((), ())),
        preferred_element_type=jnp.float32)
    cnts_ref[...] += lax.dot_general(
        oh, jnp.ones((BT, H), jnp.float32), (((0,), (0,)), ((), ())),
        preferred_element_type=jnp.float32)

    @pl.when(i == NB - 1)
    def _final():
        pooled = sums_ref[...] / jnp.maximum(cnts_ref[...], 1.0)
        o1 = jnp.maximum(
            jnp.dot(pooled, wl1_ref[...], preferred_element_type=jnp.float32)
            + bl1_ref[...], 0.0)
        o2 = (jnp.dot(o1, wl2_ref[...], preferred_element_type=jnp.float32)
              + bl2_ref[...])
        m = jnp.max(o2, axis=1, keepdims=True)
        ex = jnp.exp(o2 - m)
        lse = jnp.log(jnp.sum(ex, axis=1, keepdims=True))
        o_ref[...] = o2 - m - lse


def _pool_head(epsv, h, p0, p1, Wa, ba, Wb, bb, gam, bet,
               batch3d, Wl1, bl1, Wl2, bl2):
    return pl.pallas_call(
        _pool_body,
        grid=(NB,),
        in_specs=[
            pl.BlockSpec(memory_space=pltpu.SMEM),
            pl.BlockSpec((BT, H), lambda i: (i, 0)),
            pl.BlockSpec((BT, H), lambda i: (i, 0)),
            pl.BlockSpec((BT, H), lambda i: (i, 0)),
            pl.BlockSpec((H, H), lambda i: (0, 0)),
            pl.BlockSpec((1, H), lambda i: (0, 0)),
            pl.BlockSpec((H, H), lambda i: (0, 0)),
            pl.BlockSpec((1, H), lambda i: (0, 0)),
            pl.BlockSpec((1, H), lambda i: (0, 0)),
            pl.BlockSpec((1, H), lambda i: (0, 0)),
            pl.BlockSpec((1, 1, BT), lambda i: (i, 0, 0)),
            pl.BlockSpec((H, H), lambda i: (0, 0)),
            pl.BlockSpec((1, H), lambda i: (0, 0)),
            pl.BlockSpec((H, C), lambda i: (0, 0)),
            pl.BlockSpec((1, C), lambda i: (0, 0)),
        ],
        out_specs=pl.BlockSpec((G, C), lambda i: (0, 0)),
        out_shape=jax.ShapeDtypeStruct((G, C), jnp.float32),
        scratch_shapes=[
            pltpu.VMEM((G, H), jnp.float32),
            pltpu.VMEM((G, H), jnp.float32),
        ],
    )(epsv, h, p0, p1, Wa, ba.reshape(1, H), Wb, bb.reshape(1, H),
      gam.reshape(1, H), bet.reshape(1, H), batch3d, Wl1, bl1, Wl2, bl2)


def kernel(x, edge_index, batch, eps, Wa, ba, Wb, bb, gam, bet,
           Wl1, bl1, Wl2, bl2):
    src, dst = _pad_edges(edge_index[0], edge_index[1])
    batch3d = batch.reshape(NB, 1, BT)
    h = x
    for i in range(L - 1):
        p0, p1 = _seg_sum(h, src, dst)
        h = _dense_layer((1.0 + eps[i]).reshape(1), h, p0, p1,
                         Wa[i], ba[i], Wb[i], bb[i], gam[i], bet[i])
    p0, p1 = _seg_sum(h, src, dst)
    return _pool_head((1.0 + eps[L - 1]).reshape(1), h, p0, p1,
                      Wa[L - 1], ba[L - 1], Wb[L - 1], bb[L - 1],
                      gam[L - 1], bet[L - 1], batch3d,
                      Wl1, bl1.reshape(1, H), Wl2, bl2.reshape(1, C))
